# baseline (speedup 1.0000x reference)
import functools

import jax
import jax.numpy as jnp
from jax import lax
from jax.experimental import pallas as pl
from jax.experimental.pallas import tpu as pltpu
from jax.experimental.pallas import tpu_sc as plsc

VOCAB = 100000
EMBED_DIM = 128
BATCH = 4096
SEQ = 200

NC, NS, L = 2, 16, 16
NW = NC * NS
B_TOTAL = BATCH * SEQ
B_PER_W = B_TOTAL // NW
GRP = 128
NGRP = B_PER_W // GRP
NBUF = 4
NIT = NGRP // NBUF


@functools.partial(
    pl.kernel,
    out_type=jax.ShapeDtypeStruct((B_TOTAL, EMBED_DIM), jnp.float32),
    mesh=plsc.VectorSubcoreMesh(core_axis_name="c", subcore_axis_name="s"),
    scratch_types=[
        pltpu.VMEM((NGRP, GRP), jnp.int32),
        pltpu.VMEM((NBUF, GRP, EMBED_DIM), jnp.float32),
        pltpu.VMEM_SHARED((NS, 2, GRP, EMBED_DIM), jnp.float32),
        pltpu.SemaphoreType.DMA,
        pltpu.SemaphoreType.DMA,
        pltpu.SemaphoreType.DMA,
        pltpu.SemaphoreType.DMA,
        pltpu.SemaphoreType.DMA,
        pltpu.SemaphoreType.DMA,
        pltpu.SemaphoreType.DMA,
        pltpu.SemaphoreType.DMA,
        pltpu.SemaphoreType.DMA,
        pltpu.SemaphoreType.DMA,
    ],
)
def _gather_kernel(x_hbm, w_hbm, out_hbm, idx_v, rows_v, sp,
                   sem_g0, sem_g1, sem_g2, sem_g3,
                   sem_wd0, sem_wd1, sem_c2, sem_c3, sem_s2, sem_s3):
    sid = lax.axis_index("s")
    wid = sid * NC + lax.axis_index("c")
    base = wid * B_PER_W
    sems_g = (sem_g0, sem_g1, sem_g2, sem_g3)
    sems_wd = (sem_wd0, sem_wd1)
    sems_c = (sem_c2, sem_c3)
    sems_s = (sem_s2, sem_s3)
    pltpu.sync_copy(x_hbm.at[wid], idx_v)

    def fire_g(s, b):
        pltpu.async_copy(w_hbm.at[idx_v.at[s]], rows_v.at[b], sems_g[b])

    def wait_g(b):
        pltpu.make_async_copy(w_hbm.at[idx_v.at[0]], rows_v.at[b],
                              sems_g[b]).wait()

    def out_slice(s):
        return out_hbm.at[pl.ds(base + s * GRP, GRP)]

    def fire_wd(s, b):
        pltpu.async_copy(rows_v.at[b], out_slice(s), sems_wd[b])

    def wait_wd(b):
        pltpu.make_async_copy(rows_v.at[b], out_slice(0), sems_wd[b]).wait()

    def fire_c(b):
        pltpu.async_copy(rows_v.at[b], sp.at[sid, b - 2], sems_c[b - 2])

    def wait_c(b):
        pltpu.make_async_copy(rows_v.at[b], sp.at[sid, b - 2],
                              sems_c[b - 2]).wait()

    def fire_ws(s, b):
        pltpu.async_copy(sp.at[sid, b - 2], out_slice(s), sems_s[b - 2])

    def wait_ws(b):
        pltpu.make_async_copy(sp.at[sid, b - 2], out_slice(0),
                              sems_s[b - 2]).wait()

    for b in range(2):
        fire_g(b, b)

    def it(i, _):
        s0 = i * NBUF
        for h in range(2):
            b = 2 + h

            @pl.when(i >= 1)
            def _():
                wait_c(b)
                fire_ws(s0 + h + 2 - NBUF, b)

            @pl.when(s0 + h + 2 < NGRP)
            def _():
                fire_g(s0 + h + 2, b)

            wait_g(h)
            fire_wd(s0 + h, h)

        for h in range(2):
            b = 2 + h
            wait_wd(h)

            @pl.when(s0 + h + NBUF < NGRP)
            def _():
                fire_g(s0 + h + NBUF, h)

            wait_g(b)

            @pl.when(i >= 1)
            def _():
                wait_ws(b)

            fire_c(b)
        return 0

    lax.fori_loop(0, NIT, it, 0)
    for h in range(2):
        b = 2 + h
        wait_c(b)
        fire_ws(NGRP - 2 + h, b)
        wait_ws(b)


def kernel(x, W):
    x3 = x.reshape(NW, NGRP, GRP)
    out = _gather_kernel(x3, W)
    return out.reshape(BATCH, SEQ, EMBED_DIM)
